# baseline (speedup 1.0000x reference)
import functools

import jax
import jax.numpy as jnp
from jax.experimental import pallas as pl
from jax.experimental.pallas import tpu as pltpu

_K = 512
_ITERS = 10
_D = 256
_BLK = 512
_NBLK = 32


def _assign_block(xb, c, csq_row, xsq_col):
    s = jax.lax.dot_general(xb, c, (((1,), (1,)), ((), ())),
                            preferred_element_type=jnp.float32)
    d2 = (xsq_col - 2.0 * s) + csq_row
    m = jnp.min(d2, axis=1, keepdims=True)
    col = jax.lax.broadcasted_iota(jnp.int32, d2.shape, 1)
    idx = jnp.min(jnp.where(d2 == m, col, _K), axis=1, keepdims=True)
    return col, idx


def _assign_body(x_ref, c_ref, csq_ref, xsq_ref, asg_ref, cnt_ref):
    b = pl.program_id(0)

    @pl.when(b == 0)
    def _():
        cnt_ref[...] = jnp.zeros_like(cnt_ref)

    col, idx = _assign_block(x_ref[...], c_ref[...], csq_ref[...], xsq_ref[...])
    asg_ref[...] = idx.astype(jnp.float32)
    h = (col == idx).astype(jnp.float32)
    cnt_ref[...] += jax.lax.dot_general(
        h, jnp.ones((h.shape[0], 1), jnp.float32),
        (((0,), (0,)), ((), ())), preferred_element_type=jnp.float32)


def _chunk_boundary(q):
    r = jax.lax.rem(q, 8192)
    on = jnp.logical_or(
        r == 0,
        jnp.logical_or(
            jnp.logical_and(r <= 5600, jax.lax.rem(r, 560) == 0),
            jnp.logical_and(jnp.logical_and(r > 5600, r <= 7840),
                            jax.lax.rem(r - 5600, 448) == 0)))
    return on


def _segsum_body(asg_sm, st_sm, x_ref, sums_ref, acc_ref, run_sm):
    b = pl.program_id(0)

    @pl.when(b == 0)
    def _():
        acc_ref[...] = jnp.zeros_like(acc_ref)
        sums_ref[...] = jnp.zeros_like(sums_ref)

        def zero(i, carry):
            run_sm[i] = 0
            return carry
        jax.lax.fori_loop(0, _K, zero, 0)

    def body(j, carry):
        c = asg_sm[b * _BLK + j]
        row = x_ref[pl.ds(j, 1), :]
        gr = run_sm[c]
        run_sm[c] = gr + 1
        q = st_sm[c] + gr
        flush = jnp.logical_and(gr > 0, _chunk_boundary(q))
        msk = jnp.where(flush, 1.0, 0.0)
        a = acc_ref[pl.ds(c, 1), :]
        sums_ref[pl.ds(c, 1), :] += a * msk
        acc_ref[pl.ds(c, 1), :] = a * (1.0 - msk) + row
        return carry

    jax.lax.fori_loop(0, _BLK, body, 0)

    @pl.when(b == _NBLK - 1)
    def _():
        sums_ref[...] += acc_ref[...]


def _onehot_body(asg_ref, g_ref):
    col = jax.lax.broadcasted_iota(jnp.int32, (_BLK, _K), 1)
    idx = asg_ref[...].astype(jnp.int32)
    g_ref[...] = (col == idx).astype(jnp.float32)


def _assign_call(X, C, csq, x_sq):
    return pl.pallas_call(
        _assign_body,
        grid=(_NBLK,),
        in_specs=[pl.BlockSpec((_BLK, _D), lambda b: (b, 0)),
                  pl.BlockSpec((_K, _D), lambda b: (0, 0)),
                  pl.BlockSpec((1, _K), lambda b: (0, 0)),
                  pl.BlockSpec((_BLK, 1), lambda b: (b, 0))],
        out_specs=[pl.BlockSpec((_BLK, 1), lambda b: (b, 0)),
                   pl.BlockSpec((_K, 1), lambda b: (0, 0))],
        out_shape=[jax.ShapeDtypeStruct((X.shape[0], 1), jnp.float32),
                   jax.ShapeDtypeStruct((_K, 1), jnp.float32)],
        compiler_params=pltpu.CompilerParams(
            dimension_semantics=("arbitrary",)),
    )(X, C, csq, x_sq)


def _segsum_call(assign_i, starts_i, X):
    grid_spec = pltpu.PrefetchScalarGridSpec(
        num_scalar_prefetch=2,
        grid=(_NBLK,),
        in_specs=[pl.BlockSpec((_BLK, _D), lambda b, *_: (b, 0))],
        out_specs=pl.BlockSpec((_K, _D), lambda b, *_: (0, 0)),
        scratch_shapes=[pltpu.VMEM((_K, _D), jnp.float32),
                        pltpu.SMEM((_K,), jnp.int32)],
    )
    return pl.pallas_call(
        _segsum_body,
        grid_spec=grid_spec,
        out_shape=jax.ShapeDtypeStruct((_K, _D), jnp.float32),
        compiler_params=pltpu.CompilerParams(
            dimension_semantics=("arbitrary",)),
    )(assign_i, starts_i, X)


def _onehot_call(asg):
    return pl.pallas_call(
        _onehot_body,
        grid=(_NBLK,),
        in_specs=[pl.BlockSpec((_BLK, 1), lambda b: (b, 0))],
        out_specs=pl.BlockSpec((_BLK, _K), lambda b: (b, 0)),
        out_shape=jax.ShapeDtypeStruct((_NBLK * _BLK, _K), jnp.float32),
    )(asg)


def kernel(X):
    x_sq = (X * X).sum(axis=1, keepdims=True)
    C = X[:_K]
    asg = None
    for t in range(_ITERS):
        csq = (C * C).sum(axis=1)[None, :]
        asg, counts = _assign_call(X, C, csq, x_sq)
        if t == _ITERS - 1:
            break
        cnt_i = counts.astype(jnp.int32).ravel()
        starts = jnp.cumsum(cnt_i) - cnt_i
        sums = _segsum_call(asg.astype(jnp.int32).ravel(), starts, X)
        C = jnp.where(counts > 0.0, sums / jnp.maximum(counts, 1.0), C)
    return _onehot_call(asg)
